# baseline (speedup 1.0000x reference)
import functools

import jax
import jax.numpy as jnp
from jax import lax
from jax.experimental import pallas as pl
from jax.experimental.pallas import tpu as pltpu
from jax.experimental.pallas import tpu_sc as plsc

_H = 768
_E = 64
_TOKENS = 32768
_EPS = 1e-6
_ROOT = _H ** -0.5

_NC, _NS, _L = 2, 16, 16
_NW = _NC * _NS
_C = 2
_CT = _TOKENS // _C
_GU = 4
_UE = 3


def _logits_body(x_ref, w_ref, s_ref, lt_ref):
    x = x_ref[...]
    ms = jnp.mean(x * x, axis=1, keepdims=True)
    n = x * jax.lax.rsqrt(ms + _EPS)
    n = n * jnp.float32(_ROOT)
    n = n * s_ref[...]
    nb = n.astype(jnp.bfloat16)
    wb = w_ref[...].astype(jnp.bfloat16)
    lt_ref[...] = jax.lax.dot_general(
        wb, nb, (((1,), (1,)), ((), ())),
        preferred_element_type=jnp.float32)


_sc_mesh = plsc.VectorSubcoreMesh(core_axis_name="c", subcore_axis_name="s")


def _make_sc_topk(ct):
    tpw = ct // _NW
    nslab = max(1, tpw // 256)
    tps = tpw // nslab

    @functools.partial(
        pl.kernel,
        mesh=_sc_mesh,
        out_type=[jax.ShapeDtypeStruct((2, ct), jnp.float32),
                  jax.ShapeDtypeStruct((2, ct), jnp.int32)],
        scratch_types=[pltpu.VMEM((_E, tpw), jnp.float32),
                       pltpu.VMEM((2, tpw), jnp.float32),
                       pltpu.VMEM((2, tpw), jnp.int32),
                       pltpu.SemaphoreType.DMA((nslab,))],
    )
    def _sc_topk(lt_hbm, w_hbm, i_hbm, lt_v, w_v, i_v, sems):
        wid = lax.axis_index("s") * _NC + lax.axis_index("c")
        base = wid * tpw

        copies = [
            pltpu.async_copy(
                lt_hbm.at[:, pl.ds(base + s * tps, tps)],
                lt_v.at[:, pl.ds(s * tps, tps)],
                sems.at[s])
            for s in range(nslab)
        ]

        for s in range(nslab):
            copies[s].wait()
            for b in range(tps // (_GU * _L)):
                col0 = s * tps + b * (_GU * _L)
                cols = [col0 + u * _L for u in range(_GU)]

                def scan_e(it, carry, cols=cols):
                    out = carry
                    for k in range(_UE):
                        e = 1 + it * _UE + k
                        es = jnp.full((_L,), e, jnp.int32)
                        nxt = []
                        for u in range(_GU):
                            m1, i1, m2, i2 = out[u]
                            v = lt_v[e, pl.ds(cols[u], _L)]
                            gt1 = v > m1
                            gt2 = v > m2
                            m2n = jnp.where(gt1, m1, jnp.where(gt2, v, m2))
                            i2n = jnp.where(gt1, i1, jnp.where(gt2, es, i2))
                            m1n = jnp.where(gt1, v, m1)
                            i1n = jnp.where(gt1, es, i1)
                            nxt.append((m1n, i1n, m2n, i2n))
                        out = tuple(nxt)
                    return out

                zi = jnp.zeros((_L,), jnp.int32)
                ninf = jnp.full((_L,), -jnp.inf, jnp.float32)
                init = tuple(
                    (lt_v[0, pl.ds(cols[u], _L)], zi, ninf, zi)
                    for u in range(_GU))
                res = lax.fori_loop(0, (_E - 1) // _UE, scan_e, init)
                for u in range(_GU):
                    m1, i1, m2, i2 = res[u]
                    ex = jnp.exp(m2 - m1)
                    w2 = ex / (1.0 + ex)
                    w1 = 1.0 - w2
                    w_v[0, pl.ds(cols[u], _L)] = w1
                    w_v[1, pl.ds(cols[u], _L)] = w2
                    i_v[0, pl.ds(cols[u], _L)] = i1
                    i_v[1, pl.ds(cols[u], _L)] = i2

        pltpu.sync_copy(w_v, w_hbm.at[:, pl.ds(base, tpw)])
        pltpu.sync_copy(i_v, i_hbm.at[:, pl.ds(base, tpw)])

    return _sc_topk


_sc_topk_chunk = _make_sc_topk(_CT)


def kernel(hidden_states, W, scale):
    Tb = 2048
    scale2d = scale.reshape(1, _H)
    ws, idxs = [], []
    for c in range(_C):
        lt_c = pl.pallas_call(
            _logits_body,
            grid=(_CT // Tb,),
            in_specs=[
                pl.BlockSpec((Tb, _H),
                             functools.partial(
                                 lambda i, c: (c * (_CT // Tb) + i, 0), c=c)),
                pl.BlockSpec((_E, _H), lambda i: (0, 0)),
                pl.BlockSpec((1, _H), lambda i: (0, 0)),
            ],
            out_specs=pl.BlockSpec((_E, Tb), lambda i: (0, i)),
            out_shape=jax.ShapeDtypeStruct((_E, _CT), jnp.float32),
            compiler_params=pltpu.CompilerParams(
                dimension_semantics=("arbitrary",)),
        )(hidden_states, W, scale2d)
        w_c, i_c = _sc_topk_chunk(lt_c)
        ws.append(w_c)
        idxs.append(i_c)
    w2d = jnp.concatenate(ws, axis=1)
    i2d = jnp.concatenate(idxs, axis=1)
    return (w2d.T, i2d.T)
